# baseline (speedup 1.0000x reference)
import jax
import jax.numpy as jnp
from jax.experimental import pallas as pl
from jax.experimental.pallas import tpu as pltpu

_BM = 1024
_NBUF = 4
_N_TOKENS = 16384
_D_MODEL = 2048
_N_EXPERTS = 64


def _router_body(x_hbm, w_ref, b_ref, o_ref, xbuf, sem):
    nsteps = _N_TOKENS // _BM

    def start_copy(i):
        slot = i % _NBUF
        pltpu.make_async_copy(
            x_hbm.at[pl.ds(i * _BM, _BM), :], xbuf.at[slot], sem.at[slot]
        ).start()

    for i in range(_NBUF - 1):
        start_copy(i)
    for i in range(nsteps):
        slot = i % _NBUF
        pltpu.make_async_copy(
            x_hbm.at[pl.ds(i * _BM, _BM), :], xbuf.at[slot], sem.at[slot]
        ).wait()
        if i + _NBUF - 1 < nsteps:
            start_copy(i + _NBUF - 1)
        o_ref[pl.ds(i * _BM, _BM), :] = (
            jnp.dot(
                xbuf[slot].astype(jnp.bfloat16),
                w_ref[...].astype(jnp.bfloat16),
                preferred_element_type=jnp.float32,
            )
            + b_ref[...]
        )


@jax.jit
def kernel(x, W, b):
    wt = W.T
    b2 = b[None, :]
    return pl.pallas_call(
        _router_body,
        in_specs=[
            pl.BlockSpec(memory_space=pl.ANY),
            pl.BlockSpec(memory_space=pltpu.MemorySpace.VMEM),
            pl.BlockSpec(memory_space=pltpu.MemorySpace.VMEM),
        ],
        out_specs=pl.BlockSpec(memory_space=pltpu.MemorySpace.VMEM),
        out_shape=jax.ShapeDtypeStruct((_N_TOKENS, _N_EXPERTS), jnp.float32),
        scratch_shapes=[
            pltpu.VMEM((_NBUF, _BM, _D_MODEL), jnp.float32),
            pltpu.SemaphoreType.DMA((_NBUF,)),
        ],
    )(x, wt, b2)


# speedup vs baseline: 1.0032x; 1.0032x over previous
import jax
import jax.numpy as jnp
from jax.experimental import pallas as pl
from jax.experimental.pallas import tpu as pltpu

_BM = 512
_NBUF = 8
_N_TOKENS = 16384
_D_MODEL = 2048
_N_EXPERTS = 64


def _router_body(x_hbm, w_ref, b_ref, o_ref, xbuf, sem):
    nsteps = _N_TOKENS // _BM

    def start_copy(i):
        slot = i % _NBUF
        pltpu.make_async_copy(
            x_hbm.at[pl.ds(i * _BM, _BM), :], xbuf.at[slot], sem.at[slot]
        ).start()

    for i in range(_NBUF - 1):
        start_copy(i)
    for i in range(nsteps):
        slot = i % _NBUF
        pltpu.make_async_copy(
            x_hbm.at[pl.ds(i * _BM, _BM), :], xbuf.at[slot], sem.at[slot]
        ).wait()
        if i + _NBUF - 1 < nsteps:
            start_copy(i + _NBUF - 1)
        o_ref[pl.ds(i * _BM, _BM), :] = (
            jnp.dot(
                xbuf[slot].astype(jnp.bfloat16),
                w_ref[...].astype(jnp.bfloat16),
                preferred_element_type=jnp.float32,
            )
            + b_ref[...]
        )


@jax.jit
def kernel(x, W, b):
    wt = W.T
    b2 = b[None, :]
    return pl.pallas_call(
        _router_body,
        in_specs=[
            pl.BlockSpec(memory_space=pl.ANY),
            pl.BlockSpec(memory_space=pltpu.MemorySpace.VMEM),
            pl.BlockSpec(memory_space=pltpu.MemorySpace.VMEM),
        ],
        out_specs=pl.BlockSpec(memory_space=pltpu.MemorySpace.VMEM),
        out_shape=jax.ShapeDtypeStruct((_N_TOKENS, _N_EXPERTS), jnp.float32),
        scratch_shapes=[
            pltpu.VMEM((_NBUF, _BM, _D_MODEL), jnp.float32),
            pltpu.SemaphoreType.DMA((_NBUF,)),
        ],
    )(x, wt, b2)


# speedup vs baseline: 1.0143x; 1.0111x over previous
import jax
import jax.numpy as jnp
from jax.experimental import pallas as pl
from jax.experimental.pallas import tpu as pltpu

_BM = 512
_NBUF = 8
_N_TOKENS = 16384
_D_MODEL = 2048
_N_EXPERTS = 64


def _router_body(x_hbm, w_ref, b_ref, o_ref, *rest):
    xbufs, sem = rest[:_NBUF], rest[_NBUF]
    nsteps = _N_TOKENS // _BM

    def start_copy(i):
        slot = i % _NBUF
        pltpu.make_async_copy(
            x_hbm.at[pl.ds(i * _BM, _BM), :], xbufs[slot], sem.at[slot]
        ).start()

    for i in range(_NBUF - 1):
        start_copy(i)
    for i in range(nsteps):
        slot = i % _NBUF
        pltpu.make_async_copy(
            x_hbm.at[pl.ds(i * _BM, _BM), :], xbufs[slot], sem.at[slot]
        ).wait()
        if i + _NBUF - 1 < nsteps:
            start_copy(i + _NBUF - 1)
        o_ref[pl.ds(i * _BM, _BM), :] = (
            jnp.dot(
                xbufs[slot][...].astype(jnp.bfloat16),
                w_ref[...].astype(jnp.bfloat16),
                preferred_element_type=jnp.float32,
            )
            + b_ref[...]
        )


@jax.jit
def kernel(x, W, b):
    wt = W.T
    b2 = b[None, :]
    return pl.pallas_call(
        _router_body,
        in_specs=[
            pl.BlockSpec(memory_space=pl.ANY),
            pl.BlockSpec(memory_space=pltpu.MemorySpace.VMEM),
            pl.BlockSpec(memory_space=pltpu.MemorySpace.VMEM),
        ],
        out_specs=pl.BlockSpec(memory_space=pltpu.MemorySpace.VMEM),
        out_shape=jax.ShapeDtypeStruct((_N_TOKENS, _N_EXPERTS), jnp.float32),
        scratch_shapes=(
            [pltpu.VMEM((_BM, _D_MODEL), jnp.float32) for _ in range(_NBUF)]
            + [pltpu.SemaphoreType.DMA((_NBUF,))]
        ),
    )(x, wt, b2)
